# baseline (speedup 1.0000x reference)
import functools

import jax
import jax.numpy as jnp
from jax import lax
from jax.experimental import pallas as pl
from jax.experimental.pallas import tpu as pltpu
from jax.experimental.pallas import tpu_sc as plsc

B = 2
G = 128
C = 8
NP = 262144
TOTAL = B * NP
NW = 32
PER_W = TOTAL // NW
WPB = NW // B
CHUNK = 512
NCHUNK = PER_W // CHUNK
NGRP = CHUNK // 16
GSLICE = 128
NSLICE = CHUNK * 8 // GSLICE
ROWS = B * G * G * G

_CORNERS = [(dx, dy, dz) for dx in (0, 1) for dy in (0, 1) for dz in (0, 1)]


def _axis_split(v):
    f = v * 128.0 - 0.5
    t = f.astype(jnp.int32)
    tf = t.astype(jnp.float32)
    neg = f < tf
    i0 = jnp.where(neg, t - 1, t)
    i0f = jnp.where(neg, tf - 1.0, tf)
    frac = f - i0f
    c0 = jnp.maximum(i0, 0)
    c1 = jnp.minimum(i0 + 1, G - 1)
    return c0, c1, frac


@functools.partial(
    pl.kernel,
    mesh=plsc.VectorSubcoreMesh(core_axis_name="c", subcore_axis_name="s"),
    out_type=jax.ShapeDtypeStruct((B, NP, C), jnp.float32),
    compiler_params=pltpu.CompilerParams(
        needs_layout_passes=False, use_tc_tiling_on_sc=False
    ),
    scratch_types=[
        pltpu.VMEM((CHUNK, 4), jnp.float32),
        pltpu.VMEM((8 * CHUNK,), jnp.int32),
        pltpu.VMEM((8 * CHUNK,), jnp.float32),
        pltpu.VMEM((8 * CHUNK, C), jnp.float32),
        pltpu.VMEM((CHUNK, C), jnp.float32),
        pltpu.SemaphoreType.DMA,
    ],
)
def _sc_interp(rows_ref, locs_ref, out_ref,
               locs_v, idx_v, w_v, rows_v, out_v, sem):
    cid = lax.axis_index("c")
    sid = lax.axis_index("s")
    wid = sid * 2 + cid
    bi = wid // WPB
    pbase = (wid % WPB) * PER_W
    boff = bi * (G * G * G)
    iota = lax.iota(jnp.int32, 16)
    cols = [jnp.full((16,), c, jnp.int32) for c in range(C)]

    def chunk_body(n, carry):
        base = pbase + n * CHUNK
        pltpu.sync_copy(locs_ref.at[bi, pl.ds(base, CHUNK), :], locs_v)

        def grp_idx(j, c2):
            o = j * 16
            ro = iota + o
            x0, x1, tx = _axis_split(plsc.load_gather(locs_v, [ro, cols[0]]))
            y0, y1, ty = _axis_split(plsc.load_gather(locs_v, [ro, cols[1]]))
            z0, z1, tz = _axis_split(plsc.load_gather(locs_v, [ro, cols[2]]))
            cx = (x0 * (G * G), x1 * (G * G))
            cy = (y0 * G, y1 * G)
            cz = (z0 + boff, z1 + boff)
            ux = (1.0 - tx, tx)
            uy = (1.0 - ty, ty)
            uz = (1.0 - tz, tz)
            wxy = {(a, b): ux[a] * uy[b] for a in (0, 1) for b in (0, 1)}
            for k, (dx, dy, dz) in enumerate(_CORNERS):
                idx_v[pl.ds(k * CHUNK + o, 16)] = cx[dx] + cy[dy] + cz[dz]
                w_v[pl.ds(k * CHUNK + o, 16)] = wxy[(dx, dy)] * uz[dz]
            return c2

        lax.fori_loop(0, NGRP, grp_idx, 0)

        cps = [
            pltpu.async_copy(
                rows_ref.at[idx_v.at[pl.ds(s * GSLICE, GSLICE)]],
                rows_v.at[pl.ds(s * GSLICE, GSLICE)],
                sem,
            )
            for s in range(NSLICE)
        ]
        for cp in cps:
            cp.wait()

        def grp_acc(j, c2):
            o = j * 16
            accs = [jnp.zeros((16,), jnp.float32) for _ in range(C)]
            for k in range(8):
                w = w_v[pl.ds(k * CHUNK + o, 16)]
                ri = iota + (k * CHUNK + o)
                for c in range(C):
                    vals = plsc.load_gather(rows_v, [ri, cols[c]])
                    accs[c] = accs[c] + w * vals
            ro = iota + o
            for c in range(C):
                plsc.store_scatter(out_v, [ro, cols[c]], accs[c])
            return c2

        lax.fori_loop(0, NGRP, grp_acc, 0)
        pltpu.sync_copy(out_v, out_ref.at[bi, pl.ds(base, CHUNK), :])
        return carry

    lax.fori_loop(0, NCHUNK, chunk_body, 0)


def kernel(grid, locs):
    rows = grid.reshape(ROWS, C)
    return _sc_interp(rows, locs)


# speedup vs baseline: 6.7339x; 6.7339x over previous
import functools

import jax
import jax.numpy as jnp
from jax import lax
from jax.experimental import pallas as pl
from jax.experimental.pallas import tpu as pltpu
from jax.experimental.pallas import tpu_sc as plsc

B = 2
G = 128
C = 8
NP = 262144
TOTAL = B * NP
NW = 32
PER_W = TOTAL // NW
WPB = NW // B
CHUNK = 512
NCHUNK = PER_W // CHUNK
NBLK = CHUNK // 128
GSLICE = 128
NSLICE = CHUNK * 8 // GSLICE
ROWS = B * G * G * G

_CORNERS = [(dx, dy, dz) for dx in (0, 1) for dy in (0, 1) for dz in (0, 1)]

TCOL = B * G * G
TPW = TCOL // NW
TB = 8
NTB = TPW // TB
NSLOT = 4


@functools.partial(
    pl.kernel,
    mesh=plsc.VectorSubcoreMesh(core_axis_name="c", subcore_axis_name="s"),
    out_type=jax.ShapeDtypeStruct((ROWS * C,), jnp.float32),
    compiler_params=pltpu.CompilerParams(
        needs_layout_passes=False, use_tc_tiling_on_sc=False
    ),
    scratch_types=(
        [pltpu.VMEM((TB * 1024,), jnp.float32)] * (2 * NSLOT)
        + [pltpu.SemaphoreType.DMA] * (2 * NSLOT)
    ),
)
def _sc_transpose(g_ref, tab_ref, *bufs):
    ins = bufs[0:NSLOT]
    outs = bufs[NSLOT:2 * NSLOT]
    sis = bufs[2 * NSLOT:3 * NSLOT]
    sos = bufs[3 * NSLOT:4 * NSLOT]
    cid = lax.axis_index("c")
    sid = lax.axis_index("s")
    wid = sid * 2 + cid
    cbase = wid * TPW
    iota = lax.iota(jnp.int32, 16)
    iota8 = iota * 8

    def start_in(i, s):
        return pltpu.async_copy(
            g_ref.at[pl.ds((cbase + i * TB) * 1024, TB * 1024)], ins[s], sis[s]
        )

    def start_out(i, s):
        return pltpu.async_copy(
            outs[s], tab_ref.at[pl.ds((cbase + i * TB) * 1024, TB * 1024)],
            sos[s],
        )

    def wait_in(s):
        pltpu.make_async_copy(g_ref.at[pl.ds(0, TB * 1024)], ins[s],
                              sis[s]).wait()

    def wait_out(s):
        pltpu.make_async_copy(outs[s], tab_ref.at[pl.ds(0, TB * 1024)],
                              sos[s]).wait()

    def compute(inb, outb):
        def col_body(t, c2):
            base = t * 1024
            pbase = iota8 + base
            for c in range(8):
                for zh in range(8):
                    z0 = zh * 16
                    v = inb[pl.ds(base + c * 128 + z0, 16)]
                    plsc.store_scatter(outb, [pbase + (z0 * 8 + c)], v)
            return c2

        lax.fori_loop(0, TB, col_body, 0)

    for s in range(NSLOT):
        start_in(s, s)

    def step(ig, carry):
        for s in range(NSLOT):
            i = ig * NSLOT + s

            @pl.when(i >= NSLOT)
            def _():
                wait_out(s)

            wait_in(s)
            compute(ins[s], outs[s])
            start_out(i, s)

            @pl.when(i + NSLOT < NTB)
            def _():
                start_in(i + NSLOT, s)
        return carry

    lax.fori_loop(0, NTB // NSLOT, step, 0)
    for s in range(NSLOT):
        wait_out(s)


def _axis_split(f):
    t = f.astype(jnp.int32)
    tf = t.astype(jnp.float32)
    neg = f < tf
    i0 = jnp.where(neg, t - 1, t)
    i0f = jnp.where(neg, tf - 1.0, tf)
    frac = f - i0f
    c0 = jnp.maximum(i0, 0)
    c1 = jnp.minimum(i0 + 1, G - 1)
    return c0, c1, frac


@functools.partial(
    pl.kernel,
    mesh=plsc.VectorSubcoreMesh(core_axis_name="c", subcore_axis_name="s"),
    out_type=jax.ShapeDtypeStruct((TOTAL * C,), jnp.float32),
    compiler_params=pltpu.CompilerParams(
        needs_layout_passes=False, use_tc_tiling_on_sc=False
    ),
    scratch_types=(
        [pltpu.VMEM((4 * CHUNK,), jnp.float32)] * 2
        + [pltpu.VMEM((8 * CHUNK,), jnp.int32)] * 2
        + [pltpu.VMEM((8 * CHUNK,), jnp.float32)] * 2
        + [pltpu.VMEM((8 * CHUNK, C), jnp.float32)] * 2
        + [pltpu.VMEM((CHUNK * C,), jnp.float32)] * 2
        + [pltpu.SemaphoreType.DMA] * 4
    ),
)
def _sc_interp(rows_ref, locs_ref, out_ref, *scr):
    locs_vs = scr[0:2]
    idx_vs = scr[2:4]
    w_vs = scr[4:6]
    rows_vs = scr[6:8]
    out_vs = scr[8:10]
    gsems = scr[10:12]
    osems = scr[12:14]
    cid = lax.axis_index("c")
    sid = lax.axis_index("s")
    wid = sid * 2 + cid
    bi = wid // WPB
    pbase = (wid % WPB) * PER_W
    boff = bi * (G * G * G)
    iota = lax.iota(jnp.int32, 16)
    cols = [jnp.full((16,), c, jnp.int32) for c in range(C)]

    def stage_a(n, sl):
        base = pbase + n * CHUNK
        locs_v, idx_v, w_v, rows_v = (locs_vs[sl], idx_vs[sl], w_vs[sl],
                                      rows_vs[sl])
        pltpu.sync_copy(
            locs_ref.at[pl.ds(bi * (4 * NP) + base * 4, CHUNK * 4)], locs_v
        )

        def blk_idx(jb, c2):
            for ji in range(8):
                o = jb * 128 + ji * 16
                co = jb * 512 + ji * 16
                x = locs_v[pl.ds(co, 16)]
                y = locs_v[pl.ds(co + 128, 16)]
                z = locs_v[pl.ds(co + 256, 16)]
                x0, x1, tx = _axis_split(x * 128.0 - 0.5)
                y0, y1, ty = _axis_split(y * 128.0 - 0.5)
                z0, z1, tz = _axis_split(z * 128.0 - 0.5)
                cx = (x0 * (G * G), x1 * (G * G))
                cy = (y0 * G, y1 * G)
                cz = (z0 + boff, z1 + boff)
                ux = (1.0 - tx, tx)
                uy = (1.0 - ty, ty)
                uz = (1.0 - tz, tz)
                wxy = {(a, b): ux[a] * uy[b] for a in (0, 1) for b in (0, 1)}
                for k, (dx, dy, dz) in enumerate(_CORNERS):
                    idx_v[pl.ds(k * CHUNK + o, 16)] = cx[dx] + cy[dy] + cz[dz]
                    w_v[pl.ds(k * CHUNK + o, 16)] = wxy[(dx, dy)] * uz[dz]
            return c2

        lax.fori_loop(0, NBLK, blk_idx, 0)
        for s in range(NSLICE):
            pltpu.async_copy(
                rows_ref.at[idx_vs[sl].at[pl.ds(s * GSLICE, GSLICE)]],
                rows_v.at[pl.ds(s * GSLICE, GSLICE)],
                gsems[sl],
            )

    def stage_b(n, sl, guarded):
        base = pbase + n * CHUNK
        w_v, rows_v, out_v = w_vs[sl], rows_vs[sl], out_vs[sl]
        if guarded is not None:
            @pl.when(guarded)
            def _():
                pltpu.make_async_copy(
                    out_v,
                    out_ref.at[pl.ds(0, CHUNK * C)],
                    osems[sl],
                ).wait()
        for s in range(NSLICE):
            pltpu.make_async_copy(
                rows_ref.at[idx_vs[sl].at[pl.ds(s * GSLICE, GSLICE)]],
                rows_v.at[pl.ds(s * GSLICE, GSLICE)],
                gsems[sl],
            ).wait()

        def blk_acc(jb, c2):
            for ji in range(8):
                o = jb * 128 + ji * 16
                accs = [jnp.zeros((16,), jnp.float32) for _ in range(C)]
                for k in range(8):
                    w = w_v[pl.ds(k * CHUNK + o, 16)]
                    ri = iota + (k * CHUNK + o)
                    for c in range(C):
                        vals = plsc.load_gather(rows_v, [ri, cols[c]])
                        accs[c] = accs[c] + w * vals
                oo = jb * 1024 + ji * 16
                for c in range(C):
                    out_v[pl.ds(oo + c * 128, 16)] = accs[c]
            return c2

        lax.fori_loop(0, NBLK, blk_acc, 0)
        pltpu.async_copy(
            out_v, out_ref.at[pl.ds(bi * (C * NP) + base * C, CHUNK * C)],
            osems[sl],
        )

    stage_a(0, 0)

    def pair_body(j, carry):
        n0 = j * 2
        stage_a(n0 + 1, 1)
        stage_b(n0, 0, j > 0)

        @pl.when(j < NCHUNK // 2 - 1)
        def _():
            stage_a(n0 + 2, 0)

        stage_b(n0 + 1, 1, j > 0)
        return carry

    lax.fori_loop(0, NCHUNK // 2, pair_body, 0)
    for sl in range(2):
        pltpu.make_async_copy(
            out_vs[sl], out_ref.at[pl.ds(0, CHUNK * C)], osems[sl]
        ).wait()


def kernel(grid, locs):
    gt = grid.transpose(0, 1, 2, 4, 3).reshape(ROWS * C)
    rows = _sc_transpose(gt).reshape(ROWS, C)
    lf = (
        locs.reshape(B, NP // 128, 128, 4)
        .transpose(0, 1, 3, 2)
        .reshape(TOTAL * 4)
    )
    outf = _sc_interp(rows, lf)
    return (
        outf.reshape(B, NP // 128, C, 128)
        .transpose(0, 1, 3, 2)
        .reshape(B, NP, C)
    )
